# Initial kernel scaffold; baseline (speedup 1.0000x reference)
import jax
import jax.numpy as jnp
from jax.experimental import pallas as pl


def kernel(table, flat_indices, segment_ids):
    raise NotImplementedError("write your pallas kernel here")


# speedup vs baseline: 3.3545x; 3.3545x over previous
import functools

import jax
import jax.numpy as jnp
from jax import lax
from jax.experimental import pallas as pl
from jax.experimental.pallas import tpu as pltpu
from jax.experimental.pallas import tpu_sc as plsc

NC, NS, L = 2, 16, 16
VOCAB, DIM, TOTAL, BATCH = 100000, 128, 32768, 16
HALF = DIM // NC
G = 128
TPW = TOTAL // NS
CH = TPW // G

_mesh = plsc.VectorSubcoreMesh(core_axis_name="c", subcore_axis_name="s")


@functools.partial(
    pl.kernel,
    out_type=jax.ShapeDtypeStruct((NC, BATCH, HALF), jnp.float32),
    mesh=_mesh,
    scratch_types=[
        pltpu.VMEM((CH, G), jnp.int32),
        pltpu.VMEM((CH, G), jnp.int32),
        pltpu.VMEM((G, HALF), jnp.float32),
        pltpu.VMEM((BATCH, HALF), jnp.float32),
        pltpu.VMEM_SHARED((BATCH, HALF), jnp.float32),
        pltpu.SemaphoreType.DMA,
    ],
    compiler_params=pltpu.CompilerParams(use_tc_tiling_on_sc=False),
)
def _emb_kernel(table_hbm, idx_hbm, seg_hbm, out_hbm,
                idx_v, seg_v, rows_v, zero_v, shared_acc, sem):
    c = lax.axis_index("c")
    s = lax.axis_index("s")
    base_row = s * CH

    pltpu.sync_copy(idx_hbm.at[pl.ds(base_row, CH)], idx_v)
    pltpu.sync_copy(seg_hbm.at[pl.ds(base_row, CH)], seg_v)

    @pl.when(s == 0)
    def _():
        zero = jnp.zeros((L,), jnp.float32)
        for r in range(BATCH):
            for k in range(HALF // L):
                zero_v[r, pl.ds(k * L, L)] = zero
        pltpu.sync_copy(zero_v, shared_acc)

    for g in range(CH):
        for k in range(G // L):
            v = idx_v[g, pl.ds(k * L, L)]
            idx_v[g, pl.ds(k * L, L)] = v * NC + c

    plsc.subcore_barrier()

    for g in range(CH):
        pltpu.async_copy(table_hbm.at[idx_v.at[g]], rows_v, sem).wait()
        pltpu.sync_copy(rows_v, shared_acc.at[seg_v.at[g]], add=True)

    plsc.subcore_barrier()

    @pl.when(s == 0)
    def _():
        pltpu.sync_copy(shared_acc, out_hbm.at[c])


def kernel(table, flat_indices, segment_ids):
    table2 = table.reshape(NC * VOCAB, HALF)
    idx2 = flat_indices.reshape(TOTAL // G, G)
    seg2 = segment_ids.reshape(TOTAL // G, G)
    out = _emb_kernel(table2, idx2, seg2)
    return out.transpose(1, 0, 2).reshape(BATCH, DIM)


# speedup vs baseline: 4.2252x; 1.2596x over previous
import functools

import jax
import jax.numpy as jnp
from jax import lax
from jax.experimental import pallas as pl
from jax.experimental.pallas import tpu as pltpu
from jax.experimental.pallas import tpu_sc as plsc

NC, NS, L = 2, 16, 16
VOCAB, DIM, TOTAL, BATCH = 100000, 128, 32768, 16
HALF = DIM // NC
G = 128
TPW = TOTAL // NS
CH = TPW // G

_mesh = plsc.VectorSubcoreMesh(core_axis_name="c", subcore_axis_name="s")


@functools.partial(
    pl.kernel,
    out_type=jax.ShapeDtypeStruct((NC, BATCH, HALF), jnp.float32),
    mesh=_mesh,
    scratch_types=[
        pltpu.VMEM((CH, G), jnp.int32),
        pltpu.VMEM((CH, G), jnp.int32),
        pltpu.VMEM((G, HALF), jnp.float32),
        pltpu.VMEM((G, HALF), jnp.float32),
        pltpu.VMEM((BATCH, HALF), jnp.float32),
        pltpu.VMEM_SHARED((BATCH, HALF), jnp.float32),
        pltpu.SemaphoreType.DMA,
        pltpu.SemaphoreType.DMA,
        pltpu.SemaphoreType.DMA,
        pltpu.SemaphoreType.DMA,
    ],
    compiler_params=pltpu.CompilerParams(use_tc_tiling_on_sc=False),
)
def _emb_kernel(table_hbm, idx_hbm, seg_hbm, out_hbm,
                idx_v, seg_v, rows_a, rows_b, zero_v, shared_acc,
                gsem_a, gsem_b, ssem_a, ssem_b):
    c = lax.axis_index("c")
    s = lax.axis_index("s")
    base_row = s * CH

    pltpu.sync_copy(idx_hbm.at[pl.ds(base_row, CH)], idx_v)
    pltpu.sync_copy(seg_hbm.at[pl.ds(base_row, CH)], seg_v)

    @pl.when(s == 0)
    def _():
        zero = jnp.zeros((L,), jnp.float32)
        for r in range(BATCH):
            for k in range(HALF // L):
                zero_v[r, pl.ds(k * L, L)] = zero
        pltpu.sync_copy(zero_v, shared_acc)

    for g in range(CH):
        for k in range(G // L):
            v = idx_v[g, pl.ds(k * L, L)]
            idx_v[g, pl.ds(k * L, L)] = v * NC + c

    plsc.subcore_barrier()

    bufs, gsems, ssems = [rows_a, rows_b], [gsem_a, gsem_b], [ssem_a, ssem_b]
    gd = [None] * CH
    sd = [None] * CH
    gd[0] = pltpu.async_copy(table_hbm.at[idx_v.at[0]], bufs[0], gsems[0])
    for g in range(CH):
        b = g & 1
        if g + 1 < CH:
            if g >= 1:
                sd[g - 1].wait()
            gd[g + 1] = pltpu.async_copy(
                table_hbm.at[idx_v.at[g + 1]], bufs[1 - b], gsems[1 - b])
        gd[g].wait()
        sd[g] = pltpu.async_copy(
            bufs[b], shared_acc.at[seg_v.at[g]], ssems[b], add=True)
    sd[CH - 1].wait()
    sd[CH - 2].wait()

    plsc.subcore_barrier()

    @pl.when(s == 0)
    def _():
        pltpu.sync_copy(shared_acc, out_hbm.at[c])


def kernel(table, flat_indices, segment_ids):
    table2 = table.reshape(NC * VOCAB, HALF)
    idx2 = flat_indices.reshape(TOTAL // G, G)
    seg2 = segment_ids.reshape(TOTAL // G, G)
    out = _emb_kernel(table2, idx2, seg2)
    return out.transpose(1, 0, 2).reshape(BATCH, DIM)


# speedup vs baseline: 4.6050x; 1.0899x over previous
import functools

import jax
import jax.numpy as jnp
from jax import lax
from jax.experimental import pallas as pl
from jax.experimental.pallas import tpu as pltpu
from jax.experimental.pallas import tpu_sc as plsc

NC, NS, L = 2, 16, 16
VOCAB, DIM, TOTAL, BATCH = 100000, 128, 32768, 16
HALF = DIM // NC
G = 128
TPW = TOTAL // NS
CH = TPW // G
K = 4

_mesh = plsc.VectorSubcoreMesh(core_axis_name="c", subcore_axis_name="s")


@functools.partial(
    pl.kernel,
    out_type=jax.ShapeDtypeStruct((NC, BATCH, HALF), jnp.float32),
    mesh=_mesh,
    scratch_types=[
        pltpu.VMEM((CH, G), jnp.int32),
        pltpu.VMEM((CH, G), jnp.int32),
        [pltpu.VMEM((G, HALF), jnp.float32) for _ in range(K)],
        pltpu.VMEM((BATCH, HALF), jnp.float32),
        pltpu.VMEM_SHARED((BATCH, HALF), jnp.float32),
        [pltpu.SemaphoreType.DMA for _ in range(K)],
        [pltpu.SemaphoreType.DMA for _ in range(K)],
    ],
    compiler_params=pltpu.CompilerParams(use_tc_tiling_on_sc=False),
)
def _emb_kernel(table_hbm, idx_hbm, seg_hbm, out_hbm,
                idx_v, seg_v, bufs, zero_v, shared_acc, gsems, ssems):
    c = lax.axis_index("c")
    s = lax.axis_index("s")
    base_row = s * CH

    pltpu.sync_copy(idx_hbm.at[pl.ds(base_row, CH)], idx_v)
    pltpu.sync_copy(seg_hbm.at[pl.ds(base_row, CH)], seg_v)

    @pl.when(s == 0)
    def _():
        zero = jnp.zeros((L,), jnp.float32)
        for r in range(BATCH):
            for k in range(HALF // L):
                zero_v[r, pl.ds(k * L, L)] = zero
        pltpu.sync_copy(zero_v, shared_acc)

    @pl.loop(0, CH)
    def _(gi):
        for k in range(G // L):
            vals = idx_v[gi, pl.ds(k * L, L)]
            idx_v[gi, pl.ds(k * L, L)] = vals * NC + c

    plsc.subcore_barrier()

    gd = [None] * CH
    sd = [None] * CH
    for g in range(K):
        gd[g] = pltpu.async_copy(
            table_hbm.at[idx_v.at[g]], bufs[g], gsems[g])
    for g in range(CH):
        b = g % K
        gd[g].wait()
        sd[g] = pltpu.async_copy(
            bufs[b], shared_acc.at[seg_v.at[g]], ssems[b], add=True)
        if g + K < CH:
            sd[g].wait()
            gd[g + K] = pltpu.async_copy(
                table_hbm.at[idx_v.at[g + K]], bufs[b], gsems[b])
    for g in range(CH - K, CH):
        sd[g].wait()

    plsc.subcore_barrier()

    @pl.when(s == 0)
    def _():
        pltpu.sync_copy(shared_acc, out_hbm.at[c])


def kernel(table, flat_indices, segment_ids):
    table3 = table.reshape(NC * VOCAB, HALF)
    idx2 = flat_indices.reshape(TOTAL // G, G)
    seg2 = segment_ids.reshape(TOTAL // G, G)
    out = _emb_kernel(table3, idx2, seg2)
    return out.transpose(1, 0, 2).reshape(BATCH, DIM)


# speedup vs baseline: 4.8276x; 1.0483x over previous
import functools

import jax
import jax.numpy as jnp
from jax import lax
from jax.experimental import pallas as pl
from jax.experimental.pallas import tpu as pltpu
from jax.experimental.pallas import tpu_sc as plsc

NC, NS, L = 2, 16, 16
VOCAB, DIM, TOTAL, BATCH = 100000, 128, 32768, 16
HALF = DIM // NC
G = 128
TPW = TOTAL // NS
CH = TPW // G
K = 4

_mesh = plsc.VectorSubcoreMesh(core_axis_name="c", subcore_axis_name="s")


@functools.partial(
    pl.kernel,
    out_type=jax.ShapeDtypeStruct((BATCH, DIM), jnp.float32),
    mesh=_mesh,
    scratch_types=[
        pltpu.VMEM((CH, G), jnp.int32),
        pltpu.VMEM((CH, G), jnp.int32),
        [pltpu.VMEM((G, HALF), jnp.float32) for _ in range(K)],
        pltpu.VMEM((BATCH, HALF), jnp.float32),
        pltpu.VMEM_SHARED((BATCH, HALF), jnp.float32),
        [pltpu.SemaphoreType.DMA for _ in range(K)],
        [pltpu.SemaphoreType.DMA for _ in range(K)],
    ],
    compiler_params=pltpu.CompilerParams(use_tc_tiling_on_sc=False),
)
def _emb_kernel(table_hbm, idx_hbm, seg_hbm, out_hbm,
                idx_v, seg_v, bufs, zero_v, shared_acc, gsems, ssems):
    c = lax.axis_index("c")
    s = lax.axis_index("s")
    base_row = s * CH

    pltpu.sync_copy(idx_hbm.at[pl.ds(base_row, CH)], idx_v)
    pltpu.sync_copy(seg_hbm.at[pl.ds(base_row, CH)], seg_v)

    @pl.when(s == 0)
    def _():
        zero = jnp.zeros((L,), jnp.float32)
        for r in range(BATCH):
            for k in range(HALF // L):
                zero_v[r, pl.ds(k * L, L)] = zero
        pltpu.sync_copy(zero_v, shared_acc)

    @pl.loop(0, CH)
    def _(gi):
        for k in range(G // L):
            vals = idx_v[gi, pl.ds(k * L, L)]
            idx_v[gi, pl.ds(k * L, L)] = vals * NC + c

    plsc.subcore_barrier()

    gd = [None] * CH
    sd = [None] * CH
    for g in range(K):
        gd[g] = pltpu.async_copy(
            table_hbm.at[idx_v.at[g]], bufs[g], gsems[g])
    for g in range(CH):
        b = g % K
        gd[g].wait()
        sd[g] = pltpu.async_copy(
            bufs[b], shared_acc.at[seg_v.at[g]], ssems[b], add=True)
        if g + K < CH:
            sd[g].wait()
            gd[g + K] = pltpu.async_copy(
                table_hbm.at[idx_v.at[g + K]], bufs[b], gsems[b])
    for g in range(CH - K, CH):
        sd[g].wait()

    plsc.subcore_barrier()

    @pl.when(s == 0)
    def _():
        pltpu.sync_copy(shared_acc, out_hbm.at[:, pl.ds(c * HALF, HALF)])


def kernel(table, flat_indices, segment_ids):
    table3 = table.reshape(NC * VOCAB, HALF)
    idx2 = flat_indices.reshape(TOTAL // G, G)
    seg2 = segment_ids.reshape(TOTAL // G, G)
    return _emb_kernel(table3, idx2, seg2)
